# baseline (speedup 1.0000x reference)
import functools

import jax
import jax.numpy as jnp
from jax import lax
from jax.experimental import pallas as pl
from jax.experimental.pallas import tpu as pltpu
from jax.experimental.pallas import tpu_sc as plsc

N_NODES = 10000
N_EDGES = 160000
D_IN = 256
D_OUT = 256
DH = 128

NC = 2
NS = 16
CHUNK = 125
NCHUNK = N_EDGES // NS // CHUNK
IDXBLK = NCHUNK // 2
N_PAD = 10240
ROWS_PER_TILE = N_PAD // NS
MM_BM = 1000


def _sc_agg_body(x20, srca2d, srcb2d, dst2d, zrows, out,
                 src_v, dst_v, rows0, rows1, h_sh, sem0, sem1):
    c = lax.axis_index("c")
    s = lax.axis_index("s")

    pltpu.sync_copy(zrows, h_sh.at[pl.ds(s * ROWS_PER_TILE, ROWS_PER_TILE)])
    plsc.subcore_barrier()

    bufs = (rows0, rows1)
    sems = (sem0, sem1)

    def do_half(hb):
        base = s * NCHUNK + hb * IDXBLK

        @pl.when(c == 0)
        def _():
            pltpu.sync_copy(srca2d.at[pl.ds(base, IDXBLK)], src_v)

        @pl.when(c == 1)
        def _():
            pltpu.sync_copy(srcb2d.at[pl.ds(base, IDXBLK)], src_v)

        pltpu.sync_copy(dst2d.at[pl.ds(base, IDXBLK)], dst_v)
        pltpu.async_copy(x20.at[src_v.at[0]], rows0, sem0)
        pltpu.async_copy(x20.at[src_v.at[1]], rows1, sem1)

        def pair(i, carry):
            for t in range(2):
                j = 2 * i + t
                buf, sem = bufs[t], sems[t]
                pltpu.make_async_copy(x20.at[src_v.at[j]], buf, sem).wait()
                pltpu.sync_copy(buf, h_sh.at[dst_v.at[j]], add=True)
                nj = j + 2

                @pl.when(nj < IDXBLK)
                def _():
                    pltpu.async_copy(x20.at[src_v.at[nj]], buf, sem)
            return carry

        lax.fori_loop(0, IDXBLK // 2, pair, 0)

    for hb in range(2):
        do_half(hb)

    plsc.subcore_barrier()
    pltpu.sync_copy(h_sh.at[pl.ds(s * ROWS_PER_TILE, ROWS_PER_TILE)],
                    out.at[c, pl.ds(s * ROWS_PER_TILE, ROWS_PER_TILE)])


_sc_agg = pl.kernel(
    _sc_agg_body,
    out_type=jax.ShapeDtypeStruct((NC, N_PAD, DH), jnp.float32),
    mesh=plsc.VectorSubcoreMesh(core_axis_name="c", subcore_axis_name="s"),
    scratch_types=[
        pltpu.VMEM((IDXBLK, CHUNK), jnp.int32),
        pltpu.VMEM((IDXBLK, CHUNK), jnp.int32),
        pltpu.VMEM((CHUNK, DH), jnp.float32),
        pltpu.VMEM((CHUNK, DH), jnp.float32),
        pltpu.VMEM_SHARED((N_PAD, DH), jnp.float32),
        pltpu.SemaphoreType.DMA,
        pltpu.SemaphoreType.DMA,
    ],
)


def _mm_body(h_ref, w_ref, b_ref, o_ref):
    h = h_ref[...]
    w = w_ref[...]
    dn = (((1,), (1,)), ((), ()))
    acc = lax.dot_general(h[0], w[:, :DH], dn,
                          preferred_element_type=jnp.float32)
    acc += lax.dot_general(h[1], w[:, DH:], dn,
                           preferred_element_type=jnp.float32)
    o_ref[...] = acc + b_ref[...]


def _matmul(h, W, b2):
    return pl.pallas_call(
        _mm_body,
        grid=(N_NODES // MM_BM,),
        in_specs=[
            pl.BlockSpec((NC, MM_BM, DH), lambda i: (0, i, 0)),
            pl.BlockSpec((D_OUT, D_IN), lambda i: (0, 0)),
            pl.BlockSpec((1, D_OUT), lambda i: (0, 0)),
        ],
        out_specs=pl.BlockSpec((MM_BM, D_OUT), lambda i: (i, 0)),
        out_shape=jax.ShapeDtypeStruct((N_NODES, D_OUT), jnp.float32),
    )(h, W, b2)


def kernel(x, edge_index, W, b):
    src = edge_index[0].astype(jnp.int32)
    dst = edge_index[1].astype(jnp.int32)
    x20 = x.reshape(2 * N_NODES, DH)
    srca2d = (src * 2).reshape(NS * NCHUNK, CHUNK)
    srcb2d = (src * 2 + 1).reshape(NS * NCHUNK, CHUNK)
    dst2d = dst.reshape(NS * NCHUNK, CHUNK)
    zrows = jnp.zeros((ROWS_PER_TILE, DH), jnp.float32)
    h = _sc_agg(x20, srca2d, srcb2d, dst2d, zrows)
    return _matmul(h, W, b.reshape(1, D_OUT))


# speedup vs baseline: 1.0118x; 1.0118x over previous
import functools

import jax
import jax.numpy as jnp
from jax import lax
from jax.experimental import pallas as pl
from jax.experimental.pallas import tpu as pltpu
from jax.experimental.pallas import tpu_sc as plsc

N_NODES = 10000
N_EDGES = 160000
D_IN = 256
D_OUT = 256
DH = 128

NC = 2
NS = 16
CHUNK = 125
NCHUNK = N_EDGES // NS // CHUNK
IDXBLK = NCHUNK // 2
N_PAD = 10240
ROWS_PER_TILE = N_PAD // NS
MM_BM = 1000


def _sc_agg_body(xa, xb, src2d, dst2d, zrows, out,
                 src_v, dst_v, rows0, rows1, h_sh, g0, g1, s0, s1):
    c = lax.axis_index("c")
    s = lax.axis_index("s")

    pltpu.sync_copy(zrows, h_sh.at[pl.ds(s * ROWS_PER_TILE, ROWS_PER_TILE)])
    plsc.subcore_barrier()

    bufs = (rows0, rows1)
    gsems = (g0, g1)
    ssems = (s0, s1)

    def run(xh):
        def do_half(hb):
            base = s * NCHUNK + hb * IDXBLK
            pltpu.sync_copy(src2d.at[pl.ds(base, IDXBLK)], src_v)
            pltpu.sync_copy(dst2d.at[pl.ds(base, IDXBLK)], dst_v)
            pltpu.async_copy(xh.at[src_v.at[0]], rows0, g0)
            pltpu.async_copy(xh.at[src_v.at[1]], rows1, g1)

            def pair(i, carry):
                for t in range(2):
                    j = 2 * i + t
                    buf, gsem, ssem = bufs[t], gsems[t], ssems[t]
                    pltpu.make_async_copy(xh.at[src_v.at[j]], buf,
                                          gsem).wait()
                    pltpu.async_copy(buf, h_sh.at[dst_v.at[j]], ssem,
                                     add=True)

                    @pl.when(j + 2 < IDXBLK)
                    def _():
                        @pl.when(j >= 2)
                        def _():
                            pltpu.make_async_copy(buf, h_sh.at[dst_v.at[0]],
                                                  ssem).wait()

                        pltpu.async_copy(xh.at[src_v.at[j + 2]], buf, gsem)
                return carry

            lax.fori_loop(0, IDXBLK // 2, pair, 0)
            for t in range(2):
                for _ in range(2):
                    pltpu.make_async_copy(bufs[t], h_sh.at[dst_v.at[0]],
                                          ssems[t]).wait()

        for hb in range(2):
            do_half(hb)

    @pl.when(c == 0)
    def _():
        run(xa)

    @pl.when(c == 1)
    def _():
        run(xb)

    plsc.subcore_barrier()
    pltpu.sync_copy(h_sh.at[pl.ds(s * ROWS_PER_TILE, ROWS_PER_TILE)],
                    out.at[c, pl.ds(s * ROWS_PER_TILE, ROWS_PER_TILE)])


_sc_agg = pl.kernel(
    _sc_agg_body,
    out_type=jax.ShapeDtypeStruct((NC, N_PAD, DH), jnp.float32),
    mesh=plsc.VectorSubcoreMesh(core_axis_name="c", subcore_axis_name="s"),
    scratch_types=[
        pltpu.VMEM((IDXBLK, CHUNK), jnp.int32),
        pltpu.VMEM((IDXBLK, CHUNK), jnp.int32),
        pltpu.VMEM((CHUNK, DH), jnp.float32),
        pltpu.VMEM((CHUNK, DH), jnp.float32),
        pltpu.VMEM_SHARED((N_PAD, DH), jnp.float32),
        pltpu.SemaphoreType.DMA,
        pltpu.SemaphoreType.DMA,
        pltpu.SemaphoreType.DMA,
        pltpu.SemaphoreType.DMA,
    ],
)


def _mm_body(h_ref, w_ref, b_ref, o_ref):
    h = h_ref[...]
    w = w_ref[...]
    dn = (((1,), (1,)), ((), ()))
    acc = lax.dot_general(h[0], w[:, :DH], dn,
                          preferred_element_type=jnp.float32)
    acc += lax.dot_general(h[1], w[:, DH:], dn,
                           preferred_element_type=jnp.float32)
    o_ref[...] = acc + b_ref[...]


def _matmul(h, W, b2):
    return pl.pallas_call(
        _mm_body,
        grid=(N_NODES // MM_BM,),
        in_specs=[
            pl.BlockSpec((NC, MM_BM, DH), lambda i: (0, i, 0)),
            pl.BlockSpec((D_OUT, D_IN), lambda i: (0, 0)),
            pl.BlockSpec((1, D_OUT), lambda i: (0, 0)),
        ],
        out_specs=pl.BlockSpec((MM_BM, D_OUT), lambda i: (i, 0)),
        out_shape=jax.ShapeDtypeStruct((N_NODES, D_OUT), jnp.float32),
    )(h, W, b2)


def kernel(x, edge_index, W, b):
    src = edge_index[0].astype(jnp.int32)
    dst = edge_index[1].astype(jnp.int32)
    xa = x[:, :DH]
    xb = x[:, DH:]
    src2d = src.reshape(NS * NCHUNK, CHUNK)
    dst2d = dst.reshape(NS * NCHUNK, CHUNK)
    zrows = jnp.zeros((ROWS_PER_TILE, DH), jnp.float32)
    h = _sc_agg(xa, xb, src2d, dst2d, zrows)
    return _matmul(h, W, b.reshape(1, D_OUT))


# speedup vs baseline: 1.0515x; 1.0392x over previous
import functools

import jax
import jax.numpy as jnp
from jax import lax
from jax.experimental import pallas as pl
from jax.experimental.pallas import tpu as pltpu
from jax.experimental.pallas import tpu_sc as plsc

N_NODES = 10000
N_EDGES = 160000
D_IN = 256
D_OUT = 256
DH = 128

NC = 2
NS = 16
CHUNK = 125
NCHUNK = N_EDGES // NS // CHUNK
IDXBLK = NCHUNK // 2
N_PAD = 10240
ROWS_PER_TILE = N_PAD // NS
MM_BM = 2000


def _sc_agg_body(xa, xb, src2d, dst2d, zrows, out,
                 src_v, dst_v, rows0, rows1, h_sh, sem0, sem1, semz):
    c = lax.axis_index("c")
    s = lax.axis_index("s")

    za = pltpu.async_copy(
        zrows, h_sh.at[pl.ds(s * ROWS_PER_TILE, ROWS_PER_TILE)], semz)
    base0 = s * NCHUNK
    sa = pltpu.async_copy(src2d.at[pl.ds(base0, IDXBLK)], src_v, sem0)
    da = pltpu.async_copy(dst2d.at[pl.ds(base0, IDXBLK)], dst_v, sem1)
    sa.wait()
    da.wait()
    za.wait()
    plsc.subcore_barrier()

    bufs = (rows0, rows1)
    sems = (sem0, sem1)

    def run(xh):
        def do_half(hb, staged):
            if not staged:
                base = s * NCHUNK + hb * IDXBLK
                pltpu.sync_copy(src2d.at[pl.ds(base, IDXBLK)], src_v)
                pltpu.sync_copy(dst2d.at[pl.ds(base, IDXBLK)], dst_v)
            pltpu.async_copy(xh.at[src_v.at[0]], rows0, sem0)
            pltpu.async_copy(xh.at[src_v.at[1]], rows1, sem1)

            def pair(i, carry):
                for t in range(2):
                    j = 2 * i + t
                    buf, sem = bufs[t], sems[t]
                    pltpu.make_async_copy(xh.at[src_v.at[j]], buf,
                                          sem).wait()
                    pltpu.sync_copy(buf, h_sh.at[dst_v.at[j]], add=True)
                    nj = j + 2

                    @pl.when(nj < IDXBLK)
                    def _():
                        pltpu.async_copy(xh.at[src_v.at[nj]], buf, sem)
                return carry

            lax.fori_loop(0, IDXBLK // 2, pair, 0)

        do_half(0, staged=True)
        do_half(1, staged=False)

    @pl.when(c == 0)
    def _():
        run(xa)

    @pl.when(c == 1)
    def _():
        run(xb)

    plsc.subcore_barrier()
    pltpu.sync_copy(h_sh.at[pl.ds(s * ROWS_PER_TILE, ROWS_PER_TILE)],
                    out.at[c, pl.ds(s * ROWS_PER_TILE, ROWS_PER_TILE)])


_sc_agg = pl.kernel(
    _sc_agg_body,
    out_type=jax.ShapeDtypeStruct((NC, N_PAD, DH), jnp.float32),
    mesh=plsc.VectorSubcoreMesh(core_axis_name="c", subcore_axis_name="s"),
    scratch_types=[
        pltpu.VMEM((IDXBLK, CHUNK), jnp.int32),
        pltpu.VMEM((IDXBLK, CHUNK), jnp.int32),
        pltpu.VMEM((CHUNK, DH), jnp.float32),
        pltpu.VMEM((CHUNK, DH), jnp.float32),
        pltpu.VMEM_SHARED((N_PAD, DH), jnp.float32),
        pltpu.SemaphoreType.DMA,
        pltpu.SemaphoreType.DMA,
        pltpu.SemaphoreType.DMA,
    ],
)


def _mm_body(h_ref, w_ref, b_ref, o_ref):
    h = h_ref[...]
    w = w_ref[...]
    dn = (((1,), (1,)), ((), ()))
    acc = lax.dot_general(h[0], w[:, :DH], dn,
                          preferred_element_type=jnp.float32)
    acc += lax.dot_general(h[1], w[:, DH:], dn,
                           preferred_element_type=jnp.float32)
    o_ref[...] = acc + b_ref[...]


def _matmul(h, W, b2):
    return pl.pallas_call(
        _mm_body,
        grid=(N_NODES // MM_BM,),
        in_specs=[
            pl.BlockSpec((NC, MM_BM, DH), lambda i: (0, i, 0)),
            pl.BlockSpec((D_OUT, D_IN), lambda i: (0, 0)),
            pl.BlockSpec((1, D_OUT), lambda i: (0, 0)),
        ],
        out_specs=pl.BlockSpec((MM_BM, D_OUT), lambda i: (i, 0)),
        out_shape=jax.ShapeDtypeStruct((N_NODES, D_OUT), jnp.float32),
    )(h, W, b2)


def kernel(x, edge_index, W, b):
    src = edge_index[0].astype(jnp.int32)
    dst = edge_index[1].astype(jnp.int32)
    xa = x[:, :DH]
    xb = x[:, DH:]
    src2d = src.reshape(NS * NCHUNK, CHUNK)
    dst2d = dst.reshape(NS * NCHUNK, CHUNK)
    zrows = jnp.zeros((ROWS_PER_TILE, DH), jnp.float32)
    h = _sc_agg(xa, xb, src2d, dst2d, zrows)
    return _matmul(h, W, b.reshape(1, D_OUT))


# speedup vs baseline: 1.0557x; 1.0039x over previous
import functools

import jax
import jax.numpy as jnp
from jax import lax
from jax.experimental import pallas as pl
from jax.experimental.pallas import tpu as pltpu
from jax.experimental.pallas import tpu_sc as plsc

N_NODES = 10000
N_EDGES = 160000
D_IN = 256
D_OUT = 256
DH = 128

NC = 2
NS = 16
CHUNK = 125
NCHUNK = N_EDGES // NS // CHUNK
IDXBLK = NCHUNK // 2
N_PAD = 10240
ROWS_PER_TILE = N_PAD // NS
MM_BM = 2000


def _sc_agg_body(xa, xb, src2d, dst2d, zrows, out,
                 src_v, dst_v, rows0, rows1, h_sh, sem0, sem1, semz):
    c = lax.axis_index("c")
    s = lax.axis_index("s")

    za = pltpu.async_copy(
        zrows, h_sh.at[pl.ds(s * ROWS_PER_TILE, ROWS_PER_TILE)], semz)
    base0 = s * NCHUNK
    sa = pltpu.async_copy(src2d.at[pl.ds(base0, IDXBLK)], src_v, sem0)
    da = pltpu.async_copy(dst2d.at[pl.ds(base0, IDXBLK)], dst_v, sem1)
    sa.wait()
    da.wait()

    bufs = (rows0, rows1)
    sems = (sem0, sem1)

    def run(xh):
        def do_half(hb, staged):
            if not staged:
                base = s * NCHUNK + hb * IDXBLK
                pltpu.sync_copy(src2d.at[pl.ds(base, IDXBLK)], src_v)
                pltpu.sync_copy(dst2d.at[pl.ds(base, IDXBLK)], dst_v)
            pltpu.async_copy(xh.at[src_v.at[0]], rows0, sem0)
            pltpu.async_copy(xh.at[src_v.at[1]], rows1, sem1)
            if staged:
                za.wait()
                plsc.subcore_barrier()

            def pair(i, carry):
                for t in range(2):
                    j = 2 * i + t
                    buf, sem = bufs[t], sems[t]
                    pltpu.make_async_copy(xh.at[src_v.at[j]], buf,
                                          sem).wait()
                    pltpu.sync_copy(buf, h_sh.at[dst_v.at[j]], add=True)
                    nj = j + 2

                    @pl.when(nj < IDXBLK)
                    def _():
                        pltpu.async_copy(xh.at[src_v.at[nj]], buf, sem)
                return carry

            lax.fori_loop(0, IDXBLK // 2, pair, 0)

        do_half(0, staged=True)
        do_half(1, staged=False)

    @pl.when(c == 0)
    def _():
        run(xa)

    @pl.when(c == 1)
    def _():
        run(xb)

    plsc.subcore_barrier()
    pltpu.sync_copy(h_sh.at[pl.ds(s * ROWS_PER_TILE, ROWS_PER_TILE)],
                    out.at[c, pl.ds(s * ROWS_PER_TILE, ROWS_PER_TILE)])


_sc_agg = pl.kernel(
    _sc_agg_body,
    out_type=jax.ShapeDtypeStruct((NC, N_PAD, DH), jnp.float32),
    mesh=plsc.VectorSubcoreMesh(core_axis_name="c", subcore_axis_name="s"),
    scratch_types=[
        pltpu.VMEM((IDXBLK, CHUNK), jnp.int32),
        pltpu.VMEM((IDXBLK, CHUNK), jnp.int32),
        pltpu.VMEM((CHUNK, DH), jnp.float32),
        pltpu.VMEM((CHUNK, DH), jnp.float32),
        pltpu.VMEM_SHARED((N_PAD, DH), jnp.float32),
        pltpu.SemaphoreType.DMA,
        pltpu.SemaphoreType.DMA,
        pltpu.SemaphoreType.DMA,
    ],
)


def _mm_body(h_ref, w_ref, b_ref, o_ref):
    h = h_ref[...]
    w = w_ref[...]
    dn = (((1,), (1,)), ((), ()))
    acc = lax.dot_general(h[0], w[:, :DH], dn,
                          preferred_element_type=jnp.float32)
    acc += lax.dot_general(h[1], w[:, DH:], dn,
                           preferred_element_type=jnp.float32)
    o_ref[...] = acc + b_ref[...]


def _matmul(h, W, b2):
    return pl.pallas_call(
        _mm_body,
        grid=(N_NODES // MM_BM,),
        in_specs=[
            pl.BlockSpec((NC, MM_BM, DH), lambda i: (0, i, 0)),
            pl.BlockSpec((D_OUT, D_IN), lambda i: (0, 0)),
            pl.BlockSpec((1, D_OUT), lambda i: (0, 0)),
        ],
        out_specs=pl.BlockSpec((MM_BM, D_OUT), lambda i: (i, 0)),
        out_shape=jax.ShapeDtypeStruct((N_NODES, D_OUT), jnp.float32),
    )(h, W, b2)


def kernel(x, edge_index, W, b):
    src = edge_index[0].astype(jnp.int32)
    dst = edge_index[1].astype(jnp.int32)
    xa = x[:, :DH]
    xb = x[:, DH:]
    src2d = src.reshape(NS * NCHUNK, CHUNK)
    dst2d = dst.reshape(NS * NCHUNK, CHUNK)
    zrows = jnp.zeros((ROWS_PER_TILE, DH), jnp.float32)
    h = _sc_agg(xa, xb, src2d, dst2d, zrows)
    return _matmul(h, W, b.reshape(1, D_OUT))


# speedup vs baseline: 1.0612x; 1.0052x over previous
import jax
import jax.numpy as jnp
from jax import lax
from jax.experimental import pallas as pl
from jax.experimental.pallas import tpu as pltpu
from jax.experimental.pallas import tpu_sc as plsc

N_NODES = 10000
N_EDGES = 160000
D_IN = 256
D_OUT = 256
DH = 128

NC = 2
NS = 16
CHUNK = 125
NCHUNK = N_EDGES // NS // CHUNK
IDXBLK = NCHUNK // 2
N_PAD = 10240
ROWS_PER_TILE = N_PAD // NS
MM_BM = 2000


def _sc_agg_body(xa, xb, src2d, dst2d, zrows, out,
                 src_v, dst_v, rows0, rows1, h_sh, sem0, sem1, semz):
    c = lax.axis_index("c")
    s = lax.axis_index("s")

    za = pltpu.async_copy(
        zrows, h_sh.at[pl.ds(s * ROWS_PER_TILE, ROWS_PER_TILE)], semz)
    base0 = s * NCHUNK
    sa = pltpu.async_copy(src2d.at[pl.ds(base0, IDXBLK)], src_v, sem0)
    da = pltpu.async_copy(dst2d.at[pl.ds(base0, IDXBLK)], dst_v, sem1)
    sa.wait()
    da.wait()

    bufs = (rows0, rows1)
    sems = (sem0, sem1)

    def run(xh):
        def do_half(hb, staged):
            if not staged:
                base = s * NCHUNK + hb * IDXBLK
                pltpu.sync_copy(src2d.at[pl.ds(base, IDXBLK)], src_v)
                pltpu.sync_copy(dst2d.at[pl.ds(base, IDXBLK)], dst_v)
            pltpu.async_copy(xh.at[src_v.at[0]], rows0, sem0)
            pltpu.async_copy(xh.at[src_v.at[1]], rows1, sem1)
            if staged:
                za.wait()
                plsc.subcore_barrier()

            def pair(i, carry):
                for t in range(2):
                    j = 2 * i + t
                    buf, sem = bufs[t], sems[t]
                    pltpu.make_async_copy(xh.at[src_v.at[j]], buf,
                                          sem).wait()
                    pltpu.sync_copy(buf, h_sh.at[dst_v.at[j]], add=True)
                    nj = j + 2

                    @pl.when(nj < IDXBLK)
                    def _():
                        pltpu.async_copy(xh.at[src_v.at[nj]], buf, sem)
                return carry

            lax.fori_loop(0, IDXBLK // 2, pair, 0)

        do_half(0, staged=True)
        do_half(1, staged=False)

    @pl.when(c == 0)
    def _():
        run(xa)

    @pl.when(c == 1)
    def _():
        run(xb)

    plsc.subcore_barrier()
    pltpu.sync_copy(h_sh.at[pl.ds(s * ROWS_PER_TILE, ROWS_PER_TILE)],
                    out.at[c, pl.ds(s * ROWS_PER_TILE, ROWS_PER_TILE)])


_sc_agg = pl.kernel(
    _sc_agg_body,
    out_type=jax.ShapeDtypeStruct((NC, N_PAD, DH), jnp.float32),
    mesh=plsc.VectorSubcoreMesh(core_axis_name="c", subcore_axis_name="s"),
    scratch_types=[
        pltpu.VMEM((IDXBLK, CHUNK), jnp.int32),
        pltpu.VMEM((IDXBLK, CHUNK), jnp.int32),
        pltpu.VMEM((CHUNK, DH), jnp.float32),
        pltpu.VMEM((CHUNK, DH), jnp.float32),
        pltpu.VMEM_SHARED((N_PAD, DH), jnp.float32),
        pltpu.SemaphoreType.DMA,
        pltpu.SemaphoreType.DMA,
        pltpu.SemaphoreType.DMA,
    ],
)


def _mm_body(h_ref, w_ref, b_ref, o_ref):
    h = h_ref[...]
    w = w_ref[...]
    dn = (((1,), (1,)), ((), ()))
    acc = lax.dot_general(h[0], w[:, :DH], dn,
                          preferred_element_type=jnp.float32)
    acc += lax.dot_general(h[1], w[:, DH:], dn,
                           preferred_element_type=jnp.float32)
    o_ref[...] = acc + b_ref[...]


def _matmul(h, W, b2):
    return pl.pallas_call(
        _mm_body,
        grid=(N_NODES // MM_BM,),
        in_specs=[
            pl.BlockSpec((NC, MM_BM, DH), lambda i: (0, i, 0)),
            pl.BlockSpec((D_OUT, D_IN), lambda i: (0, 0)),
            pl.BlockSpec((1, D_OUT), lambda i: (0, 0)),
        ],
        out_specs=pl.BlockSpec((MM_BM, D_OUT), lambda i: (i, 0)),
        out_shape=jax.ShapeDtypeStruct((N_NODES, D_OUT), jnp.float32),
    )(h, W, b2)


def kernel(x, edge_index, W, b):
    src = edge_index[0].astype(jnp.int32)
    dst = edge_index[1].astype(jnp.int32)
    xa = x[:, :DH]
    xb = x[:, DH:]
    src2d = src.reshape(NS * NCHUNK, CHUNK)
    dst2d = dst.reshape(NS * NCHUNK, CHUNK)
    zrows = jnp.zeros((ROWS_PER_TILE, DH), jnp.float32)
    h = _sc_agg(xa, xb, src2d, dst2d, zrows)
    return _matmul(h, W, b.reshape(1, D_OUT))
